# Initial kernel scaffold; baseline (speedup 1.0000x reference)
import functools
import math

import jax
import jax.numpy as jnp
from jax.experimental import pallas as pl
from jax.experimental.pallas import tpu as pltpu

IN_F = 4096
OUT_F = 4096
RANK = 8
ALPHA = 32
NUM_EXPERTS = 64
TOP_K = 16
ROUTER_DIM = 16
BOTTLENECK = NUM_EXPERTS * RANK
SCALING = ALPHA / TOP_K

BT = 256

NEG_INF = jnp.float32(-jnp.inf)


def _body(x_ref, aw_ref, bw_ref, wrd_ref, wru_ref, out_ref):
    x = x_ref[...]

    s_lo = jnp.dot(x, wrd_ref[...], preferred_element_type=jnp.float32)
    scores = jnp.dot(s_lo, wru_ref[...], preferred_element_type=jnp.float32)

    thr = jnp.max(scores, axis=-1, keepdims=True)
    for _ in range(TOP_K - 1):
        below = jnp.where(scores < thr, scores, NEG_INF)
        thr = jnp.max(below, axis=-1, keepdims=True)
    rowmax = jnp.max(scores, axis=-1, keepdims=True)
    mask = scores >= thr
    p = jnp.where(mask, jnp.exp(scores - rowmax), 0.0)
    gate = p / jnp.sum(p, axis=-1, keepdims=True)

    z = jnp.dot(x, aw_ref[...], preferred_element_type=jnp.float32)

    r = jax.lax.broadcasted_iota(jnp.int32, (NUM_EXPERTS, BOTTLENECK), 0)
    c = jax.lax.broadcasted_iota(jnp.int32, (NUM_EXPERTS, BOTTLENECK), 1)
    expand = (c // RANK == r).astype(jnp.float32)
    gate_exp = jnp.dot(gate, expand, preferred_element_type=jnp.float32)

    zg = z * gate_exp
    out = jnp.dot(zg, bw_ref[...], preferred_element_type=jnp.float32)
    out_ref[...] = out * SCALING


@jax.jit
def kernel(x, A_w, B_w, Wr_down, Wr_up):
    orig_shape = x.shape
    T = math.prod(orig_shape[:-1])
    x2 = x.reshape(T, IN_F)
    grid = (T // BT,)
    out = pl.pallas_call(
        _body,
        grid=grid,
        in_specs=[
            pl.BlockSpec((BT, IN_F), lambda i: (i, 0)),
            pl.BlockSpec((IN_F, BOTTLENECK), lambda i: (0, 0)),
            pl.BlockSpec((BOTTLENECK, OUT_F), lambda i: (0, 0)),
            pl.BlockSpec((IN_F, ROUTER_DIM), lambda i: (0, 0)),
            pl.BlockSpec((ROUTER_DIM, NUM_EXPERTS), lambda i: (0, 0)),
        ],
        out_specs=pl.BlockSpec((BT, OUT_F), lambda i: (i, 0)),
        out_shape=jax.ShapeDtypeStruct((T, OUT_F), jnp.float32),
    )(x2, A_w, B_w, Wr_down, Wr_up)
    return out.reshape(*orig_shape[:-1], OUT_F)


# speedup vs baseline: 1.3903x; 1.3903x over previous
import functools
import math

import jax
import jax.numpy as jnp
from jax.experimental import pallas as pl
from jax.experimental.pallas import tpu as pltpu

IN_F = 4096
OUT_F = 4096
RANK = 8
ALPHA = 32
NUM_EXPERTS = 64
TOP_K = 16
ROUTER_DIM = 16
BOTTLENECK = NUM_EXPERTS * RANK
SCALING = ALPHA / TOP_K

BT = 256

NEG_INF = float("-inf")


def _body(x_ref, aw_ref, bw_ref, wrd_ref, wru_ref, out_ref):
    x = x_ref[...]

    s_lo = jnp.dot(x, wrd_ref[...], preferred_element_type=jnp.float32)
    scores = jnp.dot(s_lo, wru_ref[...], preferred_element_type=jnp.float32)

    thr = jnp.max(scores, axis=-1, keepdims=True)
    for _ in range(TOP_K - 1):
        below = jnp.where(scores < thr, scores, NEG_INF)
        thr = jnp.max(below, axis=-1, keepdims=True)
    rowmax = jnp.max(scores, axis=-1, keepdims=True)
    mask = scores >= thr
    p = jnp.where(mask, jnp.exp(scores - rowmax), 0.0)
    gate = p / jnp.sum(p, axis=-1, keepdims=True)

    z = jnp.dot(x, aw_ref[...], preferred_element_type=jnp.float32)

    r = jax.lax.broadcasted_iota(jnp.int32, (NUM_EXPERTS, BOTTLENECK), 0)
    c = jax.lax.broadcasted_iota(jnp.int32, (NUM_EXPERTS, BOTTLENECK), 1)
    expand = (c // RANK == r).astype(jnp.float32)
    gate_exp = jnp.dot(gate, expand, preferred_element_type=jnp.float32)

    zg = z * gate_exp
    out = jnp.dot(zg, bw_ref[...], preferred_element_type=jnp.float32)
    out_ref[...] = out * SCALING


@jax.jit
def kernel(x, A_w, B_w, Wr_down, Wr_up):
    orig_shape = x.shape
    T = math.prod(orig_shape[:-1])
    x2 = x.reshape(T, IN_F)
    grid = (T // BT,)
    out = pl.pallas_call(
        _body,
        grid=grid,
        in_specs=[
            pl.BlockSpec((BT, IN_F), lambda i: (i, 0)),
            pl.BlockSpec((IN_F, BOTTLENECK), lambda i: (0, 0)),
            pl.BlockSpec((BOTTLENECK, OUT_F), lambda i: (0, 0)),
            pl.BlockSpec((IN_F, ROUTER_DIM), lambda i: (0, 0)),
            pl.BlockSpec((ROUTER_DIM, NUM_EXPERTS), lambda i: (0, 0)),
        ],
        out_specs=pl.BlockSpec((BT, OUT_F), lambda i: (i, 0)),
        out_shape=jax.ShapeDtypeStruct((T, OUT_F), jnp.float32),
    )(x2, A_w, B_w, Wr_down, Wr_up)
    return out.reshape(*orig_shape[:-1], OUT_F)
